# baseline (speedup 1.0000x reference)
import jax
import jax.numpy as jnp
from jax.experimental import pallas as pl
from jax.experimental.pallas import tpu as pltpu

_BR = 4096


def _dense_sum_kernel(x_ref, o_ref):
    xb = x_ref[...]
    ones = jnp.ones((xb.shape[1], 128), jnp.float32)
    z = jnp.dot(xb, ones, preferred_element_type=jnp.float32)
    zv = z.reshape(_BR // 128, 128, 128)
    row = jax.lax.broadcasted_iota(jnp.int32, (128, 128), 0)
    col = jax.lax.broadcasted_iota(jnp.int32, (128, 128), 1)
    eye = (row == col).astype(jnp.float32)
    o_ref[...] = jnp.sum(zv * eye[None], axis=1)


def kernel(x):
    b, c, h, w = x.shape
    rows = b * h * w
    x2d = jnp.transpose(x, (0, 2, 3, 1)).reshape(rows, c)

    out = pl.pallas_call(
        _dense_sum_kernel,
        out_shape=jax.ShapeDtypeStruct((rows // 128, 128), x.dtype),
        grid=(rows // _BR,),
        in_specs=[pl.BlockSpec((_BR, c), lambda i: (i, 0))],
        out_specs=pl.BlockSpec((_BR // 128, 128), lambda i: (i, 0)),
        compiler_params=pltpu.CompilerParams(
            dimension_semantics=("parallel",),
            vmem_limit_bytes=64 * 1024 * 1024,
        ),
    )(x2d)
    return out.reshape(b, h, w)


# speedup vs baseline: 1.0631x; 1.0631x over previous
import jax
import jax.numpy as jnp
from jax.experimental import pallas as pl
from jax.experimental.pallas import tpu as pltpu

_BR = 16384


def _dense_sum_kernel(x_ref, o_ref):
    xb = x_ref[...]
    ones = jnp.ones((xb.shape[1], 128), jnp.float32)
    z = jnp.dot(xb, ones, preferred_element_type=jnp.float32)
    zv = z.reshape(_BR // 128, 128, 128)
    row = jax.lax.broadcasted_iota(jnp.int32, (128, 128), 0)
    col = jax.lax.broadcasted_iota(jnp.int32, (128, 128), 1)
    eye = (row == col).astype(jnp.float32)
    o_ref[...] = jnp.sum(zv * eye[None], axis=1)


def kernel(x):
    b, c, h, w = x.shape
    rows = b * h * w
    x2d = jnp.transpose(x, (0, 2, 3, 1)).reshape(rows, c)

    out = pl.pallas_call(
        _dense_sum_kernel,
        out_shape=jax.ShapeDtypeStruct((rows // 128, 128), x.dtype),
        grid=(rows // _BR,),
        in_specs=[pl.BlockSpec((_BR, c), lambda i: (i, 0))],
        out_specs=pl.BlockSpec((_BR // 128, 128), lambda i: (i, 0)),
        compiler_params=pltpu.CompilerParams(
            dimension_semantics=("parallel",),
            vmem_limit_bytes=64 * 1024 * 1024,
        ),
    )(x2d)
    return out.reshape(b, h, w)


# speedup vs baseline: 1.1392x; 1.0716x over previous
import jax
import jax.numpy as jnp
from jax.experimental import pallas as pl
from jax.experimental.pallas import tpu as pltpu

_BR = 8192


def _dense_sum_kernel(x_ref, o_ref):
    xb = x_ref[...]
    ones = jnp.ones((xb.shape[1], 128), jnp.float32)
    z = jnp.dot(xb, ones, preferred_element_type=jnp.float32)
    zv = z.reshape(_BR // 128, 128, 128)
    row = jax.lax.broadcasted_iota(jnp.int32, (128, 128), 0)
    col = jax.lax.broadcasted_iota(jnp.int32, (128, 128), 1)
    eye = (row == col).astype(jnp.float32)
    o_ref[...] = jnp.sum(zv * eye[None], axis=1)


def kernel(x):
    b, c, h, w = x.shape
    rows = b * h * w
    x2d = jnp.transpose(x, (0, 2, 3, 1)).reshape(rows, c)

    out = pl.pallas_call(
        _dense_sum_kernel,
        out_shape=jax.ShapeDtypeStruct((rows // 128, 128), x.dtype),
        grid=(rows // _BR,),
        in_specs=[pl.BlockSpec((_BR, c), lambda i: (i, 0))],
        out_specs=pl.BlockSpec((_BR // 128, 128), lambda i: (i, 0)),
        compiler_params=pltpu.CompilerParams(
            dimension_semantics=("parallel",),
            vmem_limit_bytes=64 * 1024 * 1024,
        ),
    )(x2d)
    return out.reshape(b, h, w)
